# Initial kernel scaffold; baseline (speedup 1.0000x reference)
import jax
import jax.numpy as jnp
from jax.experimental import pallas as pl


def kernel(x, edge_index, W, att_src, att_dst, bias):
    raise NotImplementedError("write your pallas kernel here")


# speedup vs baseline: 1.0698x; 1.0698x over previous
import functools

import jax
import jax.numpy as jnp
from jax import lax
from jax.experimental import pallas as pl
from jax.experimental.pallas import tpu as pltpu

N = 10000
E = 320000
F_IN = 128
F_OUT = 128

N_PAD = 10240
ROW_BLK = 512
N_BLOCKS = N_PAD // ROW_BLK
F2 = 144


def _prep_body(x_ref, w_ref, as_ref, ad_ref, h2_ref, aux_ref):
    i = pl.program_id(0)
    h = jnp.dot(x_ref[...], w_ref[...], preferred_element_type=jnp.float32)
    ones = jnp.ones((ROW_BLK, 1), jnp.float32)
    zeros = jnp.zeros((ROW_BLK, F2 - F_OUT - 1), jnp.float32)
    h2_ref[...] = jnp.concatenate([h, ones, zeros], axis=1)
    a_s = jnp.sum(h * as_ref[...], axis=1)
    a_d = jnp.sum(h * ad_ref[...], axis=1)
    row_ids = i * ROW_BLK + lax.broadcasted_iota(jnp.int32, (ROW_BLK,), 0)
    a_s = jnp.where(row_ids < N, a_s, -1e30)
    aux_ref[...] = jnp.stack([a_s, a_d], axis=0)


def _prep(x_pad, W, att_src, att_dst):
    return pl.pallas_call(
        _prep_body,
        grid=(N_BLOCKS,),
        in_specs=[
            pl.BlockSpec((ROW_BLK, F_IN), lambda i: (i, 0)),
            pl.BlockSpec((F_IN, F_OUT), lambda i: (0, 0)),
            pl.BlockSpec((1, F_OUT), lambda i: (0, 0)),
            pl.BlockSpec((1, F_OUT), lambda i: (0, 0)),
        ],
        out_specs=[
            pl.BlockSpec((ROW_BLK, F2), lambda i: (i, 0)),
            pl.BlockSpec((2, ROW_BLK), lambda i: (0, i)),
        ],
        out_shape=[
            jax.ShapeDtypeStruct((N_PAD, F2), jnp.float32),
            jax.ShapeDtypeStruct((2, N_PAD), jnp.float32),
        ],
    )(x_pad, W, att_src.reshape(1, F_OUT), att_dst.reshape(1, F_OUT))


def _finish_body(s_ref, bias_ref, out_ref):
    s = s_ref[0] + s_ref[1]
    denom = s[:, F_OUT:F_OUT + 1]
    out = s[:, :F_OUT] / (denom + 1e-16) + bias_ref[...]
    out_ref[...] = jnp.maximum(out, 0.0)


def _finish(partial, bias):
    return pl.pallas_call(
        _finish_body,
        grid=(N_BLOCKS,),
        in_specs=[
            pl.BlockSpec((2, ROW_BLK, F2), lambda i: (0, i, 0)),
            pl.BlockSpec((1, F_OUT), lambda i: (0, 0)),
        ],
        out_specs=pl.BlockSpec((ROW_BLK, F_OUT), lambda i: (i, 0)),
        out_shape=jax.ShapeDtypeStruct((N_PAD, F_OUT), jnp.float32),
    )(partial, bias.reshape(1, F_OUT))


def _edge_stage_jax(h2, aux, src, dst):
    e = aux[0, src] + aux[1, dst]
    e = jnp.where(e > 0, e, 0.2 * e)
    ex = jnp.exp(e)
    msg = h2[src] * ex[:, None]
    s = jax.ops.segment_sum(msg, dst, num_segments=N_PAD)
    return jnp.stack([s, jnp.zeros_like(s)], axis=0)


def kernel(x, edge_index, W, att_src, att_dst, bias):
    loop = jnp.arange(N, dtype=jnp.int32)
    src = jnp.concatenate([edge_index[0], loop])
    dst = jnp.concatenate([edge_index[1], loop])

    x_pad = jnp.pad(x, ((0, N_PAD - N), (0, 0)))
    h2, aux = _prep(x_pad, W, att_src, att_dst)
    partial = _edge_stage_jax(h2, aux, src, dst)
    out = _finish(partial, bias)
    return out[:N]


# speedup vs baseline: 22.2975x; 20.8433x over previous
import functools

import jax
import jax.numpy as jnp
from jax import lax
from jax.experimental import pallas as pl
from jax.experimental.pallas import tpu as pltpu
from jax.experimental.pallas import tpu_sc as plsc

N = 10000
E = 320000
F_IN = 128
F_OUT = 128

N_PAD = 10240
ROW_BLK = 512
N_BLOCKS = N_PAD // ROW_BLK
F2 = 144

NUM_TILES = 32
EB = 128
NB = 81
EPT = NB * EB
E_PAD = NUM_TILES * EPT
ROWS_PER_TILE = N_PAD // 16


def _prep_body(x_ref, w_ref, as_ref, ad_ref, h2_ref, aux_ref):
    i = pl.program_id(0)
    h = jnp.dot(x_ref[...], w_ref[...], preferred_element_type=jnp.float32)
    a_s = jnp.sum(h * as_ref[...], axis=1)
    a_d = jnp.sum(h * ad_ref[...], axis=1)
    row_ids = i * ROW_BLK + lax.broadcasted_iota(jnp.int32, (ROW_BLK,), 0)
    a_s = jnp.where(row_ids < N, a_s, -1e30)
    ones = jnp.ones((ROW_BLK, 1), jnp.float32)
    zeros = jnp.zeros((ROW_BLK, F2 - F_OUT - 2), jnp.float32)
    h2_ref[...] = jnp.concatenate([h, ones, a_s[:, None], zeros], axis=1)
    aux_ref[...] = jnp.stack([a_s, a_d], axis=0)


def _prep(x_pad, W, att_src, att_dst):
    return pl.pallas_call(
        _prep_body,
        grid=(N_BLOCKS,),
        in_specs=[
            pl.BlockSpec((ROW_BLK, F_IN), lambda i: (i, 0)),
            pl.BlockSpec((F_IN, F_OUT), lambda i: (0, 0)),
            pl.BlockSpec((1, F_OUT), lambda i: (0, 0)),
            pl.BlockSpec((1, F_OUT), lambda i: (0, 0)),
        ],
        out_specs=[
            pl.BlockSpec((ROW_BLK, F2), lambda i: (i, 0)),
            pl.BlockSpec((2, ROW_BLK), lambda i: (0, i)),
        ],
        out_shape=[
            jax.ShapeDtypeStruct((N_PAD, F2), jnp.float32),
            jax.ShapeDtypeStruct((2, N_PAD), jnp.float32),
        ],
    )(x_pad, W, att_src.reshape(1, F_OUT), att_dst.reshape(1, F_OUT))


def _finish_body(s_ref, bias_ref, out_ref):
    s = s_ref[0] + s_ref[1]
    denom = s[:, F_OUT:F_OUT + 1]
    out = s[:, :F_OUT] / (denom + 1e-16) + bias_ref[...]
    out_ref[...] = jnp.maximum(out, 0.0)


def _finish(partial, bias):
    return pl.pallas_call(
        _finish_body,
        grid=(N_BLOCKS,),
        in_specs=[
            pl.BlockSpec((2, ROW_BLK, F2), lambda i: (0, i, 0)),
            pl.BlockSpec((1, F_OUT), lambda i: (0, 0)),
        ],
        out_specs=pl.BlockSpec((ROW_BLK, F_OUT), lambda i: (i, 0)),
        out_shape=jax.ShapeDtypeStruct((N_PAD, F_OUT), jnp.float32),
    )(partial, bias.reshape(1, F_OUT))


def _edge_body(src_hbm, dst_hbm, aux_hbm, h2_hbm, out_hbm,
               src_v, dst_v, ad_v, rows_v, ex_v, s_sh):
    c = lax.axis_index("c")
    s = lax.axis_index("s")
    wid = c * 16 + s

    pltpu.sync_copy(aux_hbm.at[1], ad_v)

    def _zero_row(i, _):
        for k in range(F2 // 16):
            rows_v[i, pl.ds(k * 16, 16)] = jnp.zeros((16,), jnp.float32)
        return 0
    lax.fori_loop(0, EB, _zero_row, 0)
    for k in range(ROWS_PER_TILE // EB):
        pltpu.sync_copy(rows_v, s_sh.at[pl.ds(s * ROWS_PER_TILE + k * EB, EB)])
    plsc.subcore_barrier()

    col_as = jnp.full((16,), F_OUT + 1, jnp.int32)

    def _block(b, _):
        pltpu.sync_copy(src_hbm.at[wid, b], src_v)
        pltpu.sync_copy(dst_hbm.at[wid, b], dst_v)
        pltpu.sync_copy(h2_hbm.at[src_v], rows_v)
        for j in range(EB // 16):
            rvec = j * 16 + lax.iota(jnp.int32, 16)
            dv = dst_v[pl.ds(j * 16, 16)]
            asg = plsc.load_gather(rows_v, [rvec, col_as])
            adg = plsc.load_gather(ad_v, [dv])
            e = asg + adg
            e = jnp.where(e > 0, e, 0.2 * e)
            ex_v[pl.ds(j * 16, 16)] = jnp.exp(e)
        def _scale_grp(j, _):
            exv = ex_v[pl.ds(j * 16, 16)]
            for i in range(16):
                w = jnp.full((16,), exv[i], jnp.float32)
                r = j * 16 + i
                for k in range(F2 // 16):
                    rows_v[r, pl.ds(k * 16, 16)] = rows_v[r, pl.ds(k * 16, 16)] * w
            return 0
        lax.fori_loop(0, EB // 16, _scale_grp, 0)
        pltpu.sync_copy(rows_v, s_sh.at[dst_v], add=True)
        return 0

    lax.fori_loop(0, NB, _block, 0)
    plsc.subcore_barrier()

    for k in range(ROWS_PER_TILE // EB):
        r0 = s * ROWS_PER_TILE + k * EB
        pltpu.sync_copy(s_sh.at[pl.ds(r0, EB)], rows_v)
        pltpu.sync_copy(rows_v, out_hbm.at[c, pl.ds(r0, EB)])


_edge_kernel = functools.partial(
    pl.kernel,
    out_type=jax.ShapeDtypeStruct((2, N_PAD, F2), jnp.float32),
    mesh=plsc.VectorSubcoreMesh(core_axis_name="c", subcore_axis_name="s"),
    compiler_params=pltpu.CompilerParams(
        needs_layout_passes=False, use_tc_tiling_on_sc=False),
    scratch_types=[
        pltpu.VMEM((EB,), jnp.int32),
        pltpu.VMEM((EB,), jnp.int32),
        pltpu.VMEM((N_PAD,), jnp.float32),
        pltpu.VMEM((EB, F2), jnp.float32),
        pltpu.VMEM((EB,), jnp.float32),
        pltpu.VMEM_SHARED((N_PAD, F2), jnp.float32),
    ],
)(_edge_body)


def _edge_stage_sc(h2, aux, src, dst):
    src3 = src.reshape(NUM_TILES, NB, EB)
    dst3 = dst.reshape(NUM_TILES, NB, EB)
    return _edge_kernel(src3, dst3, aux, h2)


def kernel(x, edge_index, W, att_src, att_dst, bias):
    loop = jnp.arange(N, dtype=jnp.int32)
    pad = jnp.full((E_PAD - E - N,), N, dtype=jnp.int32)
    src = jnp.concatenate([edge_index[0], loop, pad])
    dst = jnp.concatenate([edge_index[1], loop, pad])

    x_pad = jnp.pad(x, ((0, N_PAD - N), (0, 0)))
    h2, aux = _prep(x_pad, W, att_src, att_dst)
    partial = _edge_stage_sc(h2, aux, src, dst)
    out = _finish(partial, bias)
    return out[:N]


# speedup vs baseline: 28.7372x; 1.2888x over previous
import functools

import jax
import jax.numpy as jnp
from jax import lax
from jax.experimental import pallas as pl
from jax.experimental.pallas import tpu as pltpu
from jax.experimental.pallas import tpu_sc as plsc

N = 10000
E = 320000
F_IN = 128
F_OUT = 128

N_PAD = 10240
ROW_BLK = 512
N_BLOCKS = N_PAD // ROW_BLK
F2 = 144

NUM_TILES = 32
EB = 64
NB = 162
EPT = NB * EB
E_PAD = NUM_TILES * EPT
ROWS_PER_TILE = N_PAD // 16


def _prep_body(x_ref, w_ref, as_ref, ad_ref, h2_ref, aux_ref):
    i = pl.program_id(0)
    h = jnp.dot(x_ref[...], w_ref[...], preferred_element_type=jnp.float32)
    a_s = jnp.sum(h * as_ref[...], axis=1)
    a_d = jnp.sum(h * ad_ref[...], axis=1)
    row_ids = i * ROW_BLK + lax.broadcasted_iota(jnp.int32, (ROW_BLK,), 0)
    a_s = jnp.where(row_ids < N, a_s, -1e30)
    ones = jnp.ones((ROW_BLK, 1), jnp.float32)
    zeros = jnp.zeros((ROW_BLK, F2 - F_OUT - 2), jnp.float32)
    h2_ref[...] = jnp.concatenate([h, ones, a_s[:, None], zeros], axis=1)
    aux_ref[...] = jnp.stack([a_s, a_d], axis=0)


def _prep(x_pad, W, att_src, att_dst):
    return pl.pallas_call(
        _prep_body,
        grid=(N_BLOCKS,),
        in_specs=[
            pl.BlockSpec((ROW_BLK, F_IN), lambda i: (i, 0)),
            pl.BlockSpec((F_IN, F_OUT), lambda i: (0, 0)),
            pl.BlockSpec((1, F_OUT), lambda i: (0, 0)),
            pl.BlockSpec((1, F_OUT), lambda i: (0, 0)),
        ],
        out_specs=[
            pl.BlockSpec((ROW_BLK, F2), lambda i: (i, 0)),
            pl.BlockSpec((2, ROW_BLK), lambda i: (0, i)),
        ],
        out_shape=[
            jax.ShapeDtypeStruct((N_PAD, F2), jnp.float32),
            jax.ShapeDtypeStruct((2, N_PAD), jnp.float32),
        ],
    )(x_pad, W, att_src.reshape(1, F_OUT), att_dst.reshape(1, F_OUT))


def _finish_body(s_ref, bias_ref, out_ref):
    s = s_ref[0] + s_ref[1]
    denom = s[:, F_OUT:F_OUT + 1]
    out = s[:, :F_OUT] / (denom + 1e-16) + bias_ref[...]
    out_ref[...] = jnp.maximum(out, 0.0)


def _finish(partial, bias):
    return pl.pallas_call(
        _finish_body,
        grid=(N_BLOCKS,),
        in_specs=[
            pl.BlockSpec((2, ROW_BLK, F2), lambda i: (0, i, 0)),
            pl.BlockSpec((1, F_OUT), lambda i: (0, 0)),
        ],
        out_specs=pl.BlockSpec((ROW_BLK, F_OUT), lambda i: (i, 0)),
        out_shape=jax.ShapeDtypeStruct((N_PAD, F_OUT), jnp.float32),
    )(partial, bias.reshape(1, F_OUT))


def _edge_body(src_hbm, dst_hbm, aux_hbm, h2_hbm, out_hbm,
               src_b, dst_b, ad_v, rows, sem_g, sem_s, s_sh):
    c = lax.axis_index("c")
    s = lax.axis_index("s")
    wid = c * 16 + s

    pltpu.sync_copy(aux_hbm.at[1], ad_v)

    def _zero_row(i, _):
        for k in range(F2 // 16):
            rows[0][i, pl.ds(k * 16, 16)] = jnp.zeros((16,), jnp.float32)
        return 0
    lax.fori_loop(0, EB, _zero_row, 0)
    for k in range(ROWS_PER_TILE // EB):
        pltpu.sync_copy(rows[0], s_sh.at[pl.ds(s * ROWS_PER_TILE + k * EB, EB)])
    plsc.subcore_barrier()

    col_as = jnp.full((16,), F_OUT + 1, jnp.int32)

    def _compute(q):
        def _grp(j, _):
            rvec = j * 16 + lax.iota(jnp.int32, 16)
            dv = dst_b[q][pl.ds(j * 16, 16)]
            asg = plsc.load_gather(rows[q], [rvec, col_as])
            adg = plsc.load_gather(ad_v, [dv])
            e = asg + adg
            e = jnp.where(e > 0, e, 0.2 * e)
            exv = jnp.exp(e)
            for i in range(16):
                w = jnp.full((16,), exv[i], jnp.float32)
                r = j * 16 + i
                for k in range(F2 // 16):
                    rows[q][r, pl.ds(k * 16, 16)] = rows[q][r, pl.ds(k * 16, 16)] * w
            return 0
        lax.fori_loop(0, EB // 16, _grp, 0)

    pltpu.sync_copy(src_hbm.at[wid, 0], src_b[0])
    pltpu.sync_copy(dst_hbm.at[wid, 0], dst_b[0])
    pltpu.async_copy(h2_hbm.at[src_b[0]], rows[0], sem_g)

    def _triple(g, _):
        for p in range(3):
            b = 3 * g + p
            pn = (p + 1) % 3

            @pl.when(b >= 2)
            def _():
                pltpu.make_async_copy(rows[pn], s_sh.at[dst_b[pn]], sem_s).wait()
            @pl.when(b + 1 < NB)
            def _():
                pltpu.sync_copy(src_hbm.at[wid, b + 1], src_b[pn])
                pltpu.sync_copy(dst_hbm.at[wid, b + 1], dst_b[pn])
                pltpu.async_copy(h2_hbm.at[src_b[pn]], rows[pn], sem_g)
            pltpu.make_async_copy(h2_hbm.at[src_b[p]], rows[p], sem_g).wait()
            _compute(p)
            pltpu.async_copy(rows[p], s_sh.at[dst_b[p]], sem_s, add=True)
        return 0

    lax.fori_loop(0, NB // 3, _triple, 0)
    pltpu.make_async_copy(rows[(NB - 2) % 3], s_sh.at[dst_b[(NB - 2) % 3]], sem_s).wait()
    pltpu.make_async_copy(rows[(NB - 1) % 3], s_sh.at[dst_b[(NB - 1) % 3]], sem_s).wait()
    plsc.subcore_barrier()

    for k in range(ROWS_PER_TILE // EB):
        r0 = s * ROWS_PER_TILE + k * EB
        pltpu.sync_copy(s_sh.at[pl.ds(r0, EB)], rows[0])
        pltpu.sync_copy(rows[0], out_hbm.at[c, pl.ds(r0, EB)])


_edge_kernel = functools.partial(
    pl.kernel,
    out_type=jax.ShapeDtypeStruct((2, N_PAD, F2), jnp.float32),
    mesh=plsc.VectorSubcoreMesh(core_axis_name="c", subcore_axis_name="s"),
    compiler_params=pltpu.CompilerParams(
        needs_layout_passes=False, use_tc_tiling_on_sc=False),
    scratch_types=[
        [pltpu.VMEM((EB,), jnp.int32) for _ in range(3)],
        [pltpu.VMEM((EB,), jnp.int32) for _ in range(3)],
        pltpu.VMEM((N_PAD,), jnp.float32),
        [pltpu.VMEM((EB, F2), jnp.float32) for _ in range(3)],
        pltpu.SemaphoreType.DMA,
        pltpu.SemaphoreType.DMA,
        pltpu.VMEM_SHARED((N_PAD, F2), jnp.float32),
    ],
)(_edge_body)


def _edge_stage_sc(h2, aux, src, dst):
    src3 = src.reshape(NUM_TILES, NB, EB)
    dst3 = dst.reshape(NUM_TILES, NB, EB)
    return _edge_kernel(src3, dst3, aux, h2)


def kernel(x, edge_index, W, att_src, att_dst, bias):
    loop = jnp.arange(N, dtype=jnp.int32)
    pad = jnp.full((E_PAD - E - N,), N, dtype=jnp.int32)
    src = jnp.concatenate([edge_index[0], loop, pad])
    dst = jnp.concatenate([edge_index[1], loop, pad])

    x_pad = jnp.pad(x, ((0, N_PAD - N), (0, 0)))
    h2, aux = _prep(x_pad, W, att_src, att_dst)
    partial = _edge_stage_sc(h2, aux, src, dst)
    out = _finish(partial, bias)
    return out[:N]


# speedup vs baseline: 32.1726x; 1.1195x over previous
import functools

import jax
import jax.numpy as jnp
from jax import lax
from jax.experimental import pallas as pl
from jax.experimental.pallas import tpu as pltpu
from jax.experimental.pallas import tpu_sc as plsc

N = 10000
E = 320000
F_IN = 128
F_OUT = 128

N_PAD = 10240
ROW_BLK = 512
N_BLOCKS = N_PAD // ROW_BLK
F2 = 144

NUM_TILES = 32
EB = 64
NB = 162
EPT = NB * EB
E_PAD = NUM_TILES * EPT
ROWS_PER_TILE = N_PAD // 16


def _prep_body(x_ref, w_ref, as_ref, ad_ref, h2_ref, aux_ref):
    i = pl.program_id(0)
    h = jnp.dot(x_ref[...], w_ref[...], preferred_element_type=jnp.float32)
    a_s = jnp.sum(h * as_ref[...], axis=1)
    a_d = jnp.sum(h * ad_ref[...], axis=1)
    row_ids = i * ROW_BLK + lax.broadcasted_iota(jnp.int32, (ROW_BLK,), 0)
    a_s = jnp.where(row_ids < N, a_s, -1e30)
    ones = jnp.ones((ROW_BLK, 1), jnp.float32)
    zeros = jnp.zeros((ROW_BLK, F2 - F_OUT - 2), jnp.float32)
    h2_ref[...] = jnp.concatenate([h, ones, a_s[:, None], zeros], axis=1)
    aux_ref[...] = jnp.stack([a_s, a_d], axis=0)


def _prep(x_pad, W, att_src, att_dst):
    return pl.pallas_call(
        _prep_body,
        grid=(N_BLOCKS,),
        in_specs=[
            pl.BlockSpec((ROW_BLK, F_IN), lambda i: (i, 0)),
            pl.BlockSpec((F_IN, F_OUT), lambda i: (0, 0)),
            pl.BlockSpec((1, F_OUT), lambda i: (0, 0)),
            pl.BlockSpec((1, F_OUT), lambda i: (0, 0)),
        ],
        out_specs=[
            pl.BlockSpec((ROW_BLK, F2), lambda i: (i, 0)),
            pl.BlockSpec((2, ROW_BLK), lambda i: (0, i)),
        ],
        out_shape=[
            jax.ShapeDtypeStruct((N_PAD, F2), jnp.float32),
            jax.ShapeDtypeStruct((2, N_PAD), jnp.float32),
        ],
    )(x_pad, W, att_src.reshape(1, F_OUT), att_dst.reshape(1, F_OUT))


def _finish_body(s_ref, bias_ref, out_ref):
    s = s_ref[0] + s_ref[1]
    denom = s[:, F_OUT:F_OUT + 1]
    out = s[:, :F_OUT] / (denom + 1e-16) + bias_ref[...]
    out_ref[...] = jnp.maximum(out, 0.0)


def _finish(partial, bias):
    return pl.pallas_call(
        _finish_body,
        grid=(N_BLOCKS,),
        in_specs=[
            pl.BlockSpec((2, ROW_BLK, F2), lambda i: (0, i, 0)),
            pl.BlockSpec((1, F_OUT), lambda i: (0, 0)),
        ],
        out_specs=pl.BlockSpec((ROW_BLK, F_OUT), lambda i: (i, 0)),
        out_shape=jax.ShapeDtypeStruct((N_PAD, F_OUT), jnp.float32),
    )(partial, bias.reshape(1, F_OUT))


def _edge_body(eidx_hbm, aux_hbm, h2_hbm, out_hbm,
               idx3, ad_v, rows, sem_g, sem_s, s_sh):
    c = lax.axis_index("c")
    s = lax.axis_index("s")
    wid = c * 16 + s

    pltpu.sync_copy(aux_hbm.at[1], ad_v)

    def _zero_row(i, _):
        for k in range(F2 // 16):
            rows[0][i, pl.ds(k * 16, 16)] = jnp.zeros((16,), jnp.float32)
        return 0
    lax.fori_loop(0, EB, _zero_row, 0)
    for k in range(ROWS_PER_TILE // EB):
        pltpu.sync_copy(rows[0], s_sh.at[pl.ds(s * ROWS_PER_TILE + k * EB, EB)])
    plsc.subcore_barrier()

    col_as = jnp.full((16,), F_OUT + 1, jnp.int32)

    def _compute(q, dst_ix):
        def _grp(j, _):
            rvec = j * 16 + lax.iota(jnp.int32, 16)
            dv = dst_ix[pl.ds(j * 16, 16)]
            asg = plsc.load_gather(rows[q], [rvec, col_as])
            adg = plsc.load_gather(ad_v, [dv])
            e = asg + adg
            e = jnp.where(e > 0, e, 0.2 * e)
            exv = jnp.exp(e)
            for i in range(16):
                w = jnp.full((16,), exv[i], jnp.float32)
                r = j * 16 + i
                for k in range(F2 // 16):
                    rows[q][r, pl.ds(k * 16, 16)] = rows[q][r, pl.ds(k * 16, 16)] * w
            return 0
        lax.fori_loop(0, EB // 16, _grp, 0)

    pltpu.sync_copy(eidx_hbm.at[wid, 0], idx3[0])
    pltpu.async_copy(h2_hbm.at[idx3[0].at[0, 0]], rows[0], sem_g)

    NG = NB // 3

    def _six(gg, _):
        for g2 in range(2):
            g = 2 * gg + g2
            cp, npar = g2, 1 - g2
            for p in range(3):
                b = 3 * g + p
                pn = (p + 1) % 3
                if p == 2:
                    @pl.when(g + 1 < NG)
                    def _():
                        pltpu.sync_copy(eidx_hbm.at[wid, g + 1], idx3[npar])
                wpar = cp if p == 2 else npar
                @pl.when(b >= 2)
                def _():
                    pltpu.make_async_copy(
                        rows[pn], s_sh.at[idx3[wpar].at[pn, 1]], sem_s).wait()
                nsrc = idx3[cp].at[p + 1, 0] if p < 2 else idx3[npar].at[0, 0]
                @pl.when(b + 1 < NB)
                def _():
                    pltpu.async_copy(h2_hbm.at[nsrc], rows[pn], sem_g)
                pltpu.make_async_copy(
                    h2_hbm.at[idx3[cp].at[p, 0]], rows[p], sem_g).wait()
                _compute(p, idx3[cp].at[p, 1])
                pltpu.async_copy(rows[p], s_sh.at[idx3[cp].at[p, 1]], sem_s,
                                 add=True)
        return 0

    lax.fori_loop(0, NG // 2, _six, 0)
    pltpu.make_async_copy(rows[1], s_sh.at[idx3[1].at[1, 1]], sem_s).wait()
    pltpu.make_async_copy(rows[2], s_sh.at[idx3[1].at[2, 1]], sem_s).wait()
    plsc.subcore_barrier()

    for k in range(ROWS_PER_TILE // EB):
        r0 = s * ROWS_PER_TILE + k * EB
        pltpu.sync_copy(s_sh.at[pl.ds(r0, EB)], rows[0])
        pltpu.sync_copy(rows[0], out_hbm.at[c, pl.ds(r0, EB)])


_edge_kernel = functools.partial(
    pl.kernel,
    out_type=jax.ShapeDtypeStruct((2, N_PAD, F2), jnp.float32),
    mesh=plsc.VectorSubcoreMesh(core_axis_name="c", subcore_axis_name="s"),
    compiler_params=pltpu.CompilerParams(
        needs_layout_passes=False, use_tc_tiling_on_sc=False),
    scratch_types=[
        [pltpu.VMEM((3, 2, EB), jnp.int32) for _ in range(2)],
        pltpu.VMEM((N_PAD,), jnp.float32),
        [pltpu.VMEM((EB, F2), jnp.float32) for _ in range(3)],
        pltpu.SemaphoreType.DMA,
        pltpu.SemaphoreType.DMA,
        pltpu.VMEM_SHARED((N_PAD, F2), jnp.float32),
    ],
)(_edge_body)


def _edge_stage_sc(h2, aux, src, dst):
    src3 = src.reshape(NUM_TILES, NB, EB)
    dst3 = dst.reshape(NUM_TILES, NB, EB)
    eidx = jnp.stack([src3, dst3], axis=2)
    eidx = eidx.reshape(NUM_TILES, NB // 3, 3, 2, EB)
    return _edge_kernel(eidx, aux, h2)


def kernel(x, edge_index, W, att_src, att_dst, bias):
    loop = jnp.arange(N, dtype=jnp.int32)
    pad = jnp.full((E_PAD - E - N,), N, dtype=jnp.int32)
    src = jnp.concatenate([edge_index[0], loop, pad])
    dst = jnp.concatenate([edge_index[1], loop, pad])

    x_pad = jnp.pad(x, ((0, N_PAD - N), (0, 0)))
    h2, aux = _prep(x_pad, W, att_src, att_dst)
    partial = _edge_stage_sc(h2, aux, src, dst)
    out = _finish(partial, bias)
    return out[:N]


# speedup vs baseline: 33.0361x; 1.0268x over previous
import functools

import jax
import jax.numpy as jnp
from jax import lax
from jax.experimental import pallas as pl
from jax.experimental.pallas import tpu as pltpu
from jax.experimental.pallas import tpu_sc as plsc

N = 10000
E = 320000
F_IN = 128
F_OUT = 128

N_PAD = 10240
ROW_BLK = 512
N_BLOCKS = N_PAD // ROW_BLK
F2 = 144

NUM_TILES = 32
EB = 64
NB = 162
EPT = NB * EB
E_PAD = NUM_TILES * EPT
ROWS_PER_TILE = N_PAD // 16


def _prep_body(x_ref, w_ref, as_ref, ad_ref, h2_ref, aux_ref):
    i = pl.program_id(0)
    h = jnp.dot(x_ref[...], w_ref[...], preferred_element_type=jnp.float32)
    a_s = jnp.sum(h * as_ref[...], axis=1)
    a_d = jnp.sum(h * ad_ref[...], axis=1)
    row_ids = i * ROW_BLK + lax.broadcasted_iota(jnp.int32, (ROW_BLK,), 0)
    a_s = jnp.where(row_ids < N, a_s, -1e30)
    ones = jnp.ones((ROW_BLK, 1), jnp.float32)
    zeros = jnp.zeros((ROW_BLK, F2 - F_OUT - 2), jnp.float32)
    h2_ref[...] = jnp.concatenate([h, ones, a_s[:, None], zeros], axis=1)
    aux_ref[...] = jnp.stack([a_s, a_d], axis=0)


def _prep(x_pad, W, att_src, att_dst):
    return pl.pallas_call(
        _prep_body,
        grid=(N_BLOCKS,),
        in_specs=[
            pl.BlockSpec((ROW_BLK, F_IN), lambda i: (i, 0)),
            pl.BlockSpec((F_IN, F_OUT), lambda i: (0, 0)),
            pl.BlockSpec((1, F_OUT), lambda i: (0, 0)),
            pl.BlockSpec((1, F_OUT), lambda i: (0, 0)),
        ],
        out_specs=[
            pl.BlockSpec((ROW_BLK, F2), lambda i: (i, 0)),
            pl.BlockSpec((2, ROW_BLK), lambda i: (0, i)),
        ],
        out_shape=[
            jax.ShapeDtypeStruct((N_PAD, F2), jnp.float32),
            jax.ShapeDtypeStruct((2, N_PAD), jnp.float32),
        ],
    )(x_pad, W, att_src.reshape(1, F_OUT), att_dst.reshape(1, F_OUT))


def _finish_body(s_ref, bias_ref, out_ref):
    s = s_ref[0] + s_ref[1]
    denom = s[:, F_OUT:F_OUT + 1]
    out = s[:, :F_OUT] / (denom + 1e-16) + bias_ref[...]
    out_ref[...] = jnp.maximum(out, 0.0)


def _finish(partial, bias):
    return pl.pallas_call(
        _finish_body,
        grid=(N_BLOCKS,),
        in_specs=[
            pl.BlockSpec((2, ROW_BLK, F2), lambda i: (0, i, 0)),
            pl.BlockSpec((1, F_OUT), lambda i: (0, 0)),
        ],
        out_specs=pl.BlockSpec((ROW_BLK, F_OUT), lambda i: (i, 0)),
        out_shape=jax.ShapeDtypeStruct((N_PAD, F_OUT), jnp.float32),
    )(partial, bias.reshape(1, F_OUT))


def _edge_body(eidx_hbm, aux_hbm, h2_hbm, out_hbm,
               idx3, ad_v, rows, sem_g, sem_s, sem_i, s_sh):
    c = lax.axis_index("c")
    s = lax.axis_index("s")
    wid = c * 16 + s

    pltpu.sync_copy(aux_hbm.at[1], ad_v)

    def _zero_row(i, _):
        for k in range(F2 // 16):
            rows[0][i, pl.ds(k * 16, 16)] = jnp.zeros((16,), jnp.float32)
        return 0
    lax.fori_loop(0, EB, _zero_row, 0)
    for k in range(ROWS_PER_TILE // EB):
        pltpu.sync_copy(rows[0], s_sh.at[pl.ds(s * ROWS_PER_TILE + k * EB, EB)])
    plsc.subcore_barrier()

    col_as = jnp.full((16,), F_OUT + 1, jnp.int32)

    def _compute(q, dst_ix):
        def _grp(j, _):
            rvec = j * 16 + lax.iota(jnp.int32, 16)
            dv = dst_ix[pl.ds(j * 16, 16)]
            asg = plsc.load_gather(rows[q], [rvec, col_as])
            adg = plsc.load_gather(ad_v, [dv])
            e = asg + adg
            e = jnp.where(e > 0, e, 0.2 * e)
            exv = jnp.exp(e)
            for i in range(16):
                w = jnp.full((16,), exv[i], jnp.float32)
                r = j * 16 + i
                for k in range(F2 // 16):
                    rows[q][r, pl.ds(k * 16, 16)] = rows[q][r, pl.ds(k * 16, 16)] * w
            return 0
        lax.fori_loop(0, EB // 16, _grp, 0)

    pltpu.sync_copy(eidx_hbm.at[wid, 0], idx3[0])
    pltpu.async_copy(h2_hbm.at[idx3[0].at[0, 0]], rows[0], sem_g)

    NG = NB // 3

    def _six(gg, _):
        for g2 in range(2):
            g = 2 * gg + g2
            cp, npar = g2, 1 - g2
            for p in range(3):
                b = 3 * g + p
                pn = (p + 1) % 3
                wpar = cp if p == 2 else npar
                @pl.when(b >= 2)
                def _():
                    pltpu.make_async_copy(
                        rows[pn], s_sh.at[idx3[wpar].at[pn, 1]], sem_s).wait()
                if p == 1:
                    @pl.when(g + 1 < NG)
                    def _():
                        pltpu.async_copy(eidx_hbm.at[wid, g + 1], idx3[npar],
                                         sem_i)
                if p == 2:
                    @pl.when(g + 1 < NG)
                    def _():
                        pltpu.make_async_copy(
                            eidx_hbm.at[wid, g + 1], idx3[npar], sem_i).wait()
                nsrc = idx3[cp].at[p + 1, 0] if p < 2 else idx3[npar].at[0, 0]
                @pl.when(b + 1 < NB)
                def _():
                    pltpu.async_copy(h2_hbm.at[nsrc], rows[pn], sem_g)
                pltpu.make_async_copy(
                    h2_hbm.at[idx3[cp].at[p, 0]], rows[p], sem_g).wait()
                _compute(p, idx3[cp].at[p, 1])
                pltpu.async_copy(rows[p], s_sh.at[idx3[cp].at[p, 1]], sem_s,
                                 add=True)
        return 0

    lax.fori_loop(0, NG // 2, _six, 0)
    pltpu.make_async_copy(rows[1], s_sh.at[idx3[1].at[1, 1]], sem_s).wait()
    pltpu.make_async_copy(rows[2], s_sh.at[idx3[1].at[2, 1]], sem_s).wait()
    plsc.subcore_barrier()

    for k in range(ROWS_PER_TILE // EB):
        r0 = s * ROWS_PER_TILE + k * EB
        pltpu.sync_copy(s_sh.at[pl.ds(r0, EB)], rows[0])
        pltpu.sync_copy(rows[0], out_hbm.at[c, pl.ds(r0, EB)])


_edge_kernel = functools.partial(
    pl.kernel,
    out_type=jax.ShapeDtypeStruct((2, N_PAD, F2), jnp.float32),
    mesh=plsc.VectorSubcoreMesh(core_axis_name="c", subcore_axis_name="s"),
    compiler_params=pltpu.CompilerParams(
        needs_layout_passes=False, use_tc_tiling_on_sc=False),
    scratch_types=[
        [pltpu.VMEM((3, 2, EB), jnp.int32) for _ in range(2)],
        pltpu.VMEM((N_PAD,), jnp.float32),
        [pltpu.VMEM((EB, F2), jnp.float32) for _ in range(3)],
        pltpu.SemaphoreType.DMA,
        pltpu.SemaphoreType.DMA,
        pltpu.SemaphoreType.DMA,
        pltpu.VMEM_SHARED((N_PAD, F2), jnp.float32),
    ],
)(_edge_body)


def _edge_stage_sc(h2, aux, src, dst):
    src3 = src.reshape(NUM_TILES, NB, EB)
    dst3 = dst.reshape(NUM_TILES, NB, EB)
    eidx = jnp.stack([src3, dst3], axis=2)
    eidx = eidx.reshape(NUM_TILES, NB // 3, 3, 2, EB)
    return _edge_kernel(eidx, aux, h2)


def kernel(x, edge_index, W, att_src, att_dst, bias):
    loop = jnp.arange(N, dtype=jnp.int32)
    pad = jnp.full((E_PAD - E - N,), N, dtype=jnp.int32)
    src = jnp.concatenate([edge_index[0], loop, pad])
    dst = jnp.concatenate([edge_index[1], loop, pad])

    x_pad = jnp.pad(x, ((0, N_PAD - N), (0, 0)))
    h2, aux = _prep(x_pad, W, att_src, att_dst)
    partial = _edge_stage_sc(h2, aux, src, dst)
    out = _finish(partial, bias)
    return out[:N]


# speedup vs baseline: 38.0993x; 1.1533x over previous
import functools

import jax
import jax.numpy as jnp
from jax import lax
from jax.experimental import pallas as pl
from jax.experimental.pallas import tpu as pltpu
from jax.experimental.pallas import tpu_sc as plsc

N = 10000
E = 320000
F_IN = 128
F_OUT = 128

N_PAD = 10240
ROW_BLK = 512
N_BLOCKS = N_PAD // ROW_BLK
F2 = 144

NUM_TILES = 32
EB = 64
NB = 162
EPT = NB * EB
E_PAD = NUM_TILES * EPT
ROWS_PER_TILE = N_PAD // 16


def _prep_body(x_ref, w_ref, as_ref, ad_ref, h2_ref, aux_ref):
    i = pl.program_id(0)
    h = jnp.dot(x_ref[...], w_ref[...], preferred_element_type=jnp.float32)
    a_s = jnp.sum(h * as_ref[...], axis=1)
    a_d = jnp.sum(h * ad_ref[...], axis=1)
    row_ids = i * ROW_BLK + lax.broadcasted_iota(jnp.int32, (ROW_BLK,), 0)
    a_s = jnp.where(row_ids < N, a_s, -1e30)
    ones = jnp.ones((ROW_BLK, 1), jnp.float32)
    zeros = jnp.zeros((ROW_BLK, F2 - F_OUT - 2), jnp.float32)
    h2_ref[...] = jnp.concatenate([h, ones, a_s[:, None], zeros], axis=1)
    aux_ref[...] = jnp.stack([a_s, a_d], axis=0)


def _prep(x_pad, W, att_src, att_dst):
    return pl.pallas_call(
        _prep_body,
        grid=(N_BLOCKS,),
        in_specs=[
            pl.BlockSpec((ROW_BLK, F_IN), lambda i: (i, 0)),
            pl.BlockSpec((F_IN, F_OUT), lambda i: (0, 0)),
            pl.BlockSpec((1, F_OUT), lambda i: (0, 0)),
            pl.BlockSpec((1, F_OUT), lambda i: (0, 0)),
        ],
        out_specs=[
            pl.BlockSpec((ROW_BLK, F2), lambda i: (i, 0)),
            pl.BlockSpec((2, ROW_BLK), lambda i: (0, i)),
        ],
        out_shape=[
            jax.ShapeDtypeStruct((N_PAD, F2), jnp.float32),
            jax.ShapeDtypeStruct((2, N_PAD), jnp.float32),
        ],
    )(x_pad, W, att_src.reshape(1, F_OUT), att_dst.reshape(1, F_OUT))


def _finish_body(p0_ref, p1_ref, bias_ref, out_ref):
    s = p0_ref[...] + p1_ref[...]
    denom = s[:, F_OUT:F_OUT + 1]
    out = s[:, :F_OUT] / (denom + 1e-16) + bias_ref[...]
    out_ref[...] = jnp.maximum(out, 0.0)


def _finish(p0, p1, bias):
    return pl.pallas_call(
        _finish_body,
        grid=(N_BLOCKS,),
        in_specs=[
            pl.BlockSpec((ROW_BLK, F2), lambda i: (i, 0)),
            pl.BlockSpec((ROW_BLK, F2), lambda i: (i, 0)),
            pl.BlockSpec((1, F_OUT), lambda i: (0, 0)),
        ],
        out_specs=pl.BlockSpec((ROW_BLK, F_OUT), lambda i: (i, 0)),
        out_shape=jax.ShapeDtypeStruct((N_PAD, F_OUT), jnp.float32),
    )(p0, p1, bias.reshape(1, F_OUT))


def _edge_body(eidx_hbm, aux_hbm, h2_hbm, out0_hbm, out1_hbm,
               idx3, ad_v, rows, sem_g, sem_s, sem_i, s_sh):
    c = lax.axis_index("c")
    s = lax.axis_index("s")
    wid = c * 16 + s

    pltpu.sync_copy(aux_hbm.at[1], ad_v)

    def _zero_row(i, _):
        for k in range(F2 // 16):
            rows[0][i, pl.ds(k * 16, 16)] = jnp.zeros((16,), jnp.float32)
        return 0
    lax.fori_loop(0, EB, _zero_row, 0)
    for k in range(ROWS_PER_TILE // EB):
        pltpu.sync_copy(rows[0], s_sh.at[pl.ds(s * ROWS_PER_TILE + k * EB, EB)])
    plsc.subcore_barrier()

    col_as = jnp.full((16,), F_OUT + 1, jnp.int32)

    def _compute(q, dst_ix):
        def _grp(j, _):
            rvec = j * 16 + lax.iota(jnp.int32, 16)
            dv = dst_ix[pl.ds(j * 16, 16)]
            asg = plsc.load_gather(rows[q], [rvec, col_as])
            adg = plsc.load_gather(ad_v, [dv])
            e = asg + adg
            e = jnp.where(e > 0, e, 0.2 * e)
            exv = jnp.exp(e)
            for i in range(16):
                w = jnp.full((16,), exv[i], jnp.float32)
                r = j * 16 + i
                for k in range(F2 // 16):
                    rows[q][r, pl.ds(k * 16, 16)] = rows[q][r, pl.ds(k * 16, 16)] * w
            return 0
        lax.fori_loop(0, EB // 16, _grp, 0)

    pltpu.sync_copy(eidx_hbm.at[0, wid, 0], idx3[0].at[0])
    pltpu.sync_copy(eidx_hbm.at[1, wid, 0], idx3[0].at[1])
    pltpu.async_copy(h2_hbm.at[idx3[0].at[0, 0]], rows[0], sem_g)

    NG = NB // 3

    def _six(gg, _):
        for g2 in range(2):
            g = 2 * gg + g2
            cp, npar = g2, 1 - g2
            for p in range(3):
                b = 3 * g + p
                pn = (p + 1) % 3
                wpar = cp if p == 2 else npar
                @pl.when(b >= 2)
                def _():
                    pltpu.make_async_copy(
                        rows[pn], s_sh.at[idx3[wpar].at[1, pn]], sem_s).wait()
                if p == 1:
                    @pl.when(g + 1 < NG)
                    def _():
                        pltpu.async_copy(eidx_hbm.at[0, wid, g + 1],
                                         idx3[npar].at[0], sem_i)
                        pltpu.async_copy(eidx_hbm.at[1, wid, g + 1],
                                         idx3[npar].at[1], sem_i)
                if p == 2:
                    @pl.when(g + 1 < NG)
                    def _():
                        pltpu.make_async_copy(
                            eidx_hbm.at[0, wid, g + 1], idx3[npar].at[0],
                            sem_i).wait()
                        pltpu.make_async_copy(
                            eidx_hbm.at[1, wid, g + 1], idx3[npar].at[1],
                            sem_i).wait()
                nsrc = idx3[cp].at[0, p + 1] if p < 2 else idx3[npar].at[0, 0]
                @pl.when(b + 1 < NB)
                def _():
                    pltpu.async_copy(h2_hbm.at[nsrc], rows[pn], sem_g)
                pltpu.make_async_copy(
                    h2_hbm.at[idx3[cp].at[0, p]], rows[p], sem_g).wait()
                _compute(p, idx3[cp].at[1, p])
                pltpu.async_copy(rows[p], s_sh.at[idx3[cp].at[1, p]], sem_s,
                                 add=True)
        return 0

    lax.fori_loop(0, NG // 2, _six, 0)
    pltpu.make_async_copy(rows[1], s_sh.at[idx3[1].at[1, 1]], sem_s).wait()
    pltpu.make_async_copy(rows[2], s_sh.at[idx3[1].at[1, 2]], sem_s).wait()
    plsc.subcore_barrier()

    for k in range(ROWS_PER_TILE // EB):
        r0 = s * ROWS_PER_TILE + k * EB
        pltpu.sync_copy(s_sh.at[pl.ds(r0, EB)], rows[0])

        @pl.when(c == 0)
        def _():
            pltpu.sync_copy(rows[0], out0_hbm.at[pl.ds(r0, EB)])

        @pl.when(c == 1)
        def _():
            pltpu.sync_copy(rows[0], out1_hbm.at[pl.ds(r0, EB)])


_edge_kernel = functools.partial(
    pl.kernel,
    out_type=[jax.ShapeDtypeStruct((N_PAD, F2), jnp.float32),
              jax.ShapeDtypeStruct((N_PAD, F2), jnp.float32)],
    mesh=plsc.VectorSubcoreMesh(core_axis_name="c", subcore_axis_name="s"),
    compiler_params=pltpu.CompilerParams(
        needs_layout_passes=False, use_tc_tiling_on_sc=False),
    scratch_types=[
        [pltpu.VMEM((2, 3, EB), jnp.int32) for _ in range(2)],
        pltpu.VMEM((N_PAD,), jnp.float32),
        [pltpu.VMEM((EB, F2), jnp.float32) for _ in range(3)],
        pltpu.SemaphoreType.DMA,
        pltpu.SemaphoreType.DMA,
        pltpu.SemaphoreType.DMA,
        pltpu.VMEM_SHARED((N_PAD, F2), jnp.float32),
    ],
)(_edge_body)


def _edge_stage_sc(h2, aux, eidx):
    return _edge_kernel(eidx, aux, h2)


def kernel(x, edge_index, W, att_src, att_dst, bias):
    loop = jnp.arange(N, dtype=jnp.int32)
    pad = jnp.full((E_PAD - E - N,), N, dtype=jnp.int32)
    tail = jnp.stack([jnp.concatenate([loop, pad])] * 2)
    eidx = jnp.concatenate([edge_index, tail], axis=1)
    eidx = eidx.reshape(2, NUM_TILES, NB // 3, 3, EB)

    x_pad = jnp.pad(x, ((0, N_PAD - N), (0, 0)))
    h2, aux = _prep(x_pad, W, att_src, att_dst)
    p0, p1 = _edge_stage_sc(h2, aux, eidx)
    out = _finish(p0, p1, bias)
    return out[:N]
